# Initial kernel scaffold; baseline (speedup 1.0000x reference)
import jax
import jax.numpy as jnp
from jax.experimental import pallas as pl


def kernel(h, e, dirn, diff, dist, edge_index, spatial_mask, A_w, A_b, B_w, B_b, C_w, C_b, D_w, D_b, E_w, E_b, g_h, b_h, g_e, b_e):
    raise NotImplementedError("write your pallas kernel here")


# speedup vs baseline: 3.4777x; 3.4777x over previous
import dataclasses
import functools

import jax
import jax.numpy as jnp
from jax import lax
from jax.experimental import pallas as pl
from jax.experimental.pallas import tpu as pltpu
from jax.experimental.pallas import tpu_sc as plsc

N = 10000
E = 320000
D = 128
NC = 2
NS = 16
NW = NC * NS
CH = 80
WSRC = 2 * D
N2 = 10240
ZR = N2 // NS
EBLK = 512
E1 = 192000
E2 = E - E1


def _sc_params():
    cp = pltpu.CompilerParams()
    if "needs_layout_passes" in pltpu.CompilerParams.__dataclass_fields__:
        cp = dataclasses.replace(cp, needs_layout_passes=False)
    return cp


def _mm(x, w):
    return lax.dot_general(x, w, (((1,), (1,)), ((), ())),
                           preferred_element_type=jnp.float32)


def _tables_body(h_ref, aw, ab, bw, bb, dw, db, ew, eb,
                 ah_ref, tsrc_ref, dh_ref):
    h = h_ref[...]
    ah_ref[...] = _mm(h, aw[...]) + ab[...][None, :]
    bh = _mm(h, bw[...]) + bb[...][None, :]
    ehn = _mm(h, ew[...]) + eb[...][None, :]
    dh_ref[...] = _mm(h, dw[...]) + db[...][None, :]
    tsrc_ref[...] = jnp.concatenate([bh, ehn], axis=1)


def _make_tables(h, aw, ab, bw, bb, dw, db, ew, eb):
    nb = 10
    rb = N // nb
    wspec = pl.BlockSpec((D, D), lambda i: (0, 0))
    bspec = pl.BlockSpec((D,), lambda i: (0,))
    return pl.pallas_call(
        _tables_body,
        grid=(nb,),
        in_specs=[
            pl.BlockSpec((rb, D), lambda i: (i, 0)),
            wspec, bspec, wspec, bspec, wspec, bspec, wspec, bspec,
        ],
        out_specs=[
            pl.BlockSpec((rb, D), lambda i: (i, 0)),
            pl.BlockSpec((rb, WSRC), lambda i: (i, 0)),
            pl.BlockSpec((rb, D), lambda i: (i, 0)),
        ],
        out_shape=[
            jax.ShapeDtypeStruct((N, D), jnp.float32),
            jax.ShapeDtypeStruct((N, WSRC), jnp.float32),
            jax.ShapeDtypeStruct((N, D), jnp.float32),
        ],
    )(h, aw, ab, bw, bb, dw, db, ew, eb)


def _gather_sc(tsrc, dh, src, dst, dirn_flat, aux5, eo, en):
    d0a, d1a, d2a, dsa, mka = aux5
    mesh = plsc.VectorSubcoreMesh(core_axis_name="c", subcore_axis_name="s")
    epw = en // NW
    nch = epw // CH

    @functools.partial(
        pl.kernel,
        out_type=(jax.ShapeDtypeStruct((en, WSRC), jnp.float32),
                  jax.ShapeDtypeStruct((en, D), jnp.float32),
                  jax.ShapeDtypeStruct((en,), jnp.float32)),
        mesh=mesh,
        scratch_types=[
            pltpu.VMEM((epw,), jnp.int32),
            pltpu.VMEM((epw,), jnp.int32),
            pltpu.VMEM((CH, WSRC), jnp.float32),
            pltpu.VMEM((CH, WSRC), jnp.float32),
            pltpu.VMEM((CH, D), jnp.float32),
            pltpu.VMEM((CH, D), jnp.float32),
            pltpu.VMEM((4 * N,), jnp.float32),
        ] + [pltpu.VMEM((CH,), jnp.float32) for _ in range(12)] + [
            pltpu.SemaphoreType.DMA,
            pltpu.SemaphoreType.DMA,
            pltpu.SemaphoreType.DMA,
            pltpu.SemaphoreType.DMA,
        ],
        compiler_params=_sc_params(),
    )
    def k(tsrc_hbm, dh_hbm, src_hbm, dst_hbm, dirn_hbm,
          d0_hbm, d1_hbm, d2_hbm, ds_hbm, mk_hbm,
          gsrc_hbm, gdst_hbm, am_hbm,
          si_v, di_v, rs0_v, rs1_v, rd0_v, rd1_v, dirn_v,
          x00, x01, x02, x03, x04, x05, x10, x11, x12, x13, x14, x15,
          sem_g0, sem_g1, sem_o0, sem_o1):
        wid = lax.axis_index("c") * NS + lax.axis_index("s")
        base0 = eo + wid * epw
        rs_v = (rs0_v, rs1_v)
        rd_v = (rd0_v, rd1_v)
        aux_v = ((x00, x01, x02, x03, x04), (x10, x11, x12, x13, x14))
        am_v = (x05, x15)
        sem_g = (sem_g0, sem_g1)
        sem_o = (sem_o0, sem_o1)

        pltpu.sync_copy(src_hbm.at[pl.ds(base0, epw)], si_v)
        pltpu.sync_copy(dst_hbm.at[pl.ds(base0, epw)], di_v)
        pltpu.sync_copy(dirn_hbm, dirn_v)

        def gather_descs(kk, b):
            loc = kk * CH
            base = base0 + loc
            descs = [
                pltpu.make_async_copy(
                    tsrc_hbm.at[si_v.at[pl.ds(loc, CH)]], rs_v[b],
                    sem_g[b]),
                pltpu.make_async_copy(
                    dh_hbm.at[di_v.at[pl.ds(loc, CH)]], rd_v[b],
                    sem_g[b]),
            ]
            for c, ref in enumerate((d0_hbm, d1_hbm, d2_hbm, ds_hbm,
                                     mk_hbm)):
                descs.append(pltpu.make_async_copy(
                    ref.at[pl.ds(base, CH)], aux_v[b][c], sem_g[b]))
            return descs

        def out_descs(kk, b):
            base = base0 + kk * CH
            return (
                pltpu.make_async_copy(
                    rs_v[b], gsrc_hbm.at[pl.ds(base - eo, CH)], sem_o[b]),
                pltpu.make_async_copy(
                    rd_v[b], gdst_hbm.at[pl.ds(base - eo, CH)], sem_o[b]),
                pltpu.make_async_copy(
                    am_v[b], am_hbm.at[pl.ds(base - eo, CH)], sem_o[b]),
            )

        def compute_am(kk, b):
            loc = kk * CH
            d0v, d1v, d2v, dsv, mkv = aux_v[b]
            for g in range(CH // 16):
                s16 = si_v[pl.ds(loc + g * 16, 16)] * 4
                d16 = di_v[pl.ds(loc + g * 16, 16)] * 4
                s0 = plsc.load_gather(dirn_v, [s16])
                s1 = plsc.load_gather(dirn_v, [s16 + 1])
                s2 = plsc.load_gather(dirn_v, [s16 + 2])
                t0 = plsc.load_gather(dirn_v, [d16])
                t1 = plsc.load_gather(dirn_v, [d16 + 1])
                t2 = plsc.load_gather(dirn_v, [d16 + 2])
                f0 = d0v[pl.ds(g * 16, 16)]
                f1 = d1v[pl.ds(g * 16, 16)]
                f2 = d2v[pl.ds(g * 16, 16)]
                ds = dsv[pl.ds(g * 16, 16)]
                mk = mkv[pl.ds(g * 16, 16)]
                pos = ds > 0.0
                e0 = jnp.where(pos, f0 / ds, f0)
                e1 = jnp.where(pos, f1 / ds, f1)
                e2 = jnp.where(pos, f2 / ds, f2)
                uv = s0 * t0 + s1 * t1 + s2 * t2
                ehd = e0 * t0 + e1 * t1 + e2 * t2
                a = 0.5 * (1.0 - ehd) * uv
                am_v[b][pl.ds(g * 16, 16)] = jnp.where(mk == 1.0, a, 1.0)

        for c in gather_descs(0, 0):
            c.start()

        def body(kk, b):
            @pl.when(kk + 1 < nch)
            def _():
                @pl.when(kk >= 1)
                def _():
                    for c in out_descs(kk - 1, 1 - b):
                        c.wait()
                for c in gather_descs(kk + 1, 1 - b):
                    c.start()

            for c in gather_descs(kk, b):
                c.wait()
            compute_am(kk, b)
            for c in out_descs(kk, b):
                c.start()

        @pl.loop(0, nch // 2)
        def _(t):
            body(2 * t, 0)
            body(2 * t + 1, 1)

        if nch % 2 == 1:
            body(nch - 1, 0)
        for c in out_descs(nch - 1, (nch - 1) % 2):
            c.wait()
        for c in out_descs(nch - 2, (nch - 2) % 2):
            c.wait()

    return k(tsrc, dh, src, dst, dirn_flat, d0a, d1a, d2a, dsa, mka)


def _edge_body(e_ref, gsrc_ref, gdst_ref, a_ref, cw, cb,
               enew_ref, hsig_ref, stats_ref):
    i = pl.program_id(0)
    e_blk = e_ref[...]
    ce = _mm(e_blk, cw[...]) + cb[...][None, :]
    gsrc = gsrc_ref[...]
    bh = gsrc[:, 0:D]
    eh = gsrc[:, D:2 * D]
    dh = gdst_ref[...]
    q = dh - eh
    e_new = q + ce
    he = bh + q
    norm2 = jnp.sum(e_new * e_new, axis=1, keepdims=True)
    sigma = jnp.exp(-0.5 * jnp.sqrt(norm2))
    am = jnp.transpose(a_ref[...].reshape(1, EBLK), (1, 0))
    w = am * sigma
    enew_ref[...] = e_new
    hsig_ref[...] = he * w

    blk_stats = jnp.concatenate(
        [jnp.sum(e_new, axis=0, keepdims=True),
         jnp.sum(e_new * e_new, axis=0, keepdims=True),
         jnp.zeros((6, D), jnp.float32)], axis=0)

    @pl.when(i == 0)
    def _():
        stats_ref[...] = jnp.zeros_like(stats_ref)

    stats_ref[...] += blk_stats


def _edge_pass(e, gsrc, gdst, a3, cw, cb, eob, nb):
    en = nb * EBLK
    return pl.pallas_call(
        _edge_body,
        grid=(nb,),
        in_specs=[
            pl.BlockSpec((EBLK, D), lambda i: (i + eob, 0)),
            pl.BlockSpec((EBLK, WSRC), lambda i: (i, 0)),
            pl.BlockSpec((EBLK, D), lambda i: (i, 0)),
            pl.BlockSpec((1, 1, EBLK), lambda i: (i, 0, 0)),
            pl.BlockSpec((D, D), lambda i: (0, 0)),
            pl.BlockSpec((D,), lambda i: (0,)),
        ],
        out_specs=[
            pl.BlockSpec((EBLK, D), lambda i: (i, 0)),
            pl.BlockSpec((EBLK, D), lambda i: (i, 0)),
            pl.BlockSpec((8, D), lambda i: (0, 0)),
        ],
        out_shape=[
            jax.ShapeDtypeStruct((en, D), jnp.float32),
            jax.ShapeDtypeStruct((en, D), jnp.float32),
            jax.ShapeDtypeStruct((8, D), jnp.float32),
        ],
    )(e, gsrc, gdst, a3, cw, cb)


def _scatter_sc(hsig, dst, eo, en):
    mesh = plsc.VectorSubcoreMesh(core_axis_name="c", subcore_axis_name="s",
                                  num_cores=1)
    epw = en // NS
    nch = epw // CH

    @functools.partial(
        pl.kernel,
        out_type=jax.ShapeDtypeStruct((N2, D), jnp.float32),
        mesh=mesh,
        scratch_types=[
            pltpu.VMEM((1, CH), jnp.int32),
            pltpu.VMEM((1, CH), jnp.int32),
            pltpu.VMEM((CH, D), jnp.float32),
            pltpu.VMEM((CH, D), jnp.float32),
            pltpu.VMEM_SHARED((N2, D), jnp.float32),
            pltpu.SemaphoreType.DMA,
            pltpu.SemaphoreType.DMA,
            pltpu.SemaphoreType.DMA,
            pltpu.SemaphoreType.DMA,
        ],
    )
    def k(hs_hbm, dst_hbm, out_hbm, di0_v, di1_v, rows0_v, rows1_v, acc_sh,
          sem_l0, sem_l1, sem_s0, sem_s1):
        sid = lax.axis_index("s")
        di_v = (di0_v, di1_v)
        rows_v = (rows0_v, rows1_v)
        sem_l = (sem_l0, sem_l1)
        sem_s = (sem_s0, sem_s1)

        @pl.loop(0, CH)
        def _(r):
            for g in range(D // 16):
                rows0_v[pl.ds(r, 1), pl.ds(g * 16, 16)] = (
                    jnp.zeros((1, 16), jnp.float32))

        for j in range(ZR // CH):
            pltpu.sync_copy(rows0_v, acc_sh.at[pl.ds(sid * ZR + j * CH, CH)])
        plsc.subcore_barrier()

        base0 = sid * epw

        def load_descs(kk, b):
            base = base0 + kk * CH
            return (
                pltpu.make_async_copy(
                    dst_hbm.at[pl.ds(eo + base, CH)], di_v[b].at[0],
                    sem_l[b]),
                pltpu.make_async_copy(
                    hs_hbm.at[pl.ds(base, CH)], rows_v[b], sem_l[b]),
            )

        def scat_wait(b):
            pltpu.make_async_copy(
                rows_v[b], acc_sh.at[di_v[b].at[0]], sem_s[b]).wait()

        for c in load_descs(0, 0):
            c.start()

        def body(kk, b):
            @pl.when(kk + 1 < nch)
            def _():
                @pl.when(kk >= 1)
                def _():
                    scat_wait(1 - b)
                for c in load_descs(kk + 1, 1 - b):
                    c.start()

            for c in load_descs(kk, b):
                c.wait()
            pltpu.async_copy(rows_v[b], acc_sh.at[di_v[b].at[0]], sem_s[b],
                             add=True)

        @pl.loop(0, nch // 2)
        def _(t):
            body(2 * t, 0)
            body(2 * t + 1, 1)

        scat_wait((nch - 1) % 2)
        scat_wait((nch - 2) % 2)
        plsc.subcore_barrier()

        for j in range(ZR // CH):
            row0 = sid * ZR + j * CH
            pltpu.sync_copy(acc_sh.at[pl.ds(row0, CH)], rows0_v)
            pltpu.sync_copy(rows0_v, out_hbm.at[pl.ds(row0, CH)])

    return k(hsig, dst)


def _hfin_body(h_ref, ah_ref, acca_ref, accb_ref, gh, bhp, out_ref):
    x = ah_ref[...] + acca_ref[0:N, :] + accb_ref[0:N, :]
    mu = jnp.mean(x, axis=0, keepdims=True)
    xc = x - mu
    var = jnp.mean(xc * xc, axis=0, keepdims=True)
    y = xc * lax.rsqrt(var + 1e-5) * gh[...][None, :] + bhp[...][None, :]
    out_ref[...] = h_ref[...] + jnp.maximum(y, 0.0)


def _h_finalize(h, ah, acca, accb, gh, bh):
    return pl.pallas_call(
        _hfin_body,
        out_shape=jax.ShapeDtypeStruct((N, D), jnp.float32),
    )(h, ah, acca, accb, gh, bh)


def _efin_body_first(e_ref, enew_ref, sa_ref, sb_ref, ge, be, out_ref):
    _efin_common(e_ref, enew_ref, sa_ref, sb_ref, ge, be, out_ref)


def _efin_body_chain(prev_ref, e_ref, enew_ref, sa_ref, sb_ref, ge, be,
                     out_ref):
    del prev_ref
    _efin_common(e_ref, enew_ref, sa_ref, sb_ref, ge, be, out_ref)


def _efin_common(e_ref, enew_ref, sa_ref, sb_ref, ge, be, out_ref):
    s = sa_ref[0:1, :] + sb_ref[0:1, :]
    s2 = sa_ref[1:2, :] + sb_ref[1:2, :]
    mu = s / float(E)
    var = s2 / float(E) - mu * mu
    alpha = ge[...][None, :] * lax.rsqrt(var + 1e-5)
    beta = be[...][None, :] - mu * alpha
    y = enew_ref[...] * alpha + beta
    out_ref[...] = e_ref[...] + jnp.maximum(y, 0.0)


def _e_finalize(prev, e, enew, sa, sb, ge, be, eob, nb):
    especs = [
        pl.BlockSpec((EBLK, D), lambda i: (i + eob, 0)),
        pl.BlockSpec((EBLK, D), lambda i: (i, 0)),
        pl.BlockSpec((8, D), lambda i: (0, 0)),
        pl.BlockSpec((8, D), lambda i: (0, 0)),
        pl.BlockSpec((D,), lambda i: (0,)),
        pl.BlockSpec((D,), lambda i: (0,)),
    ]
    if prev is None:
        return pl.pallas_call(
            _efin_body_first,
            grid=(nb,),
            in_specs=especs,
            out_specs=pl.BlockSpec((EBLK, D), lambda i: (i + eob, 0)),
            out_shape=jax.ShapeDtypeStruct((E, D), jnp.float32),
        )(e, enew, sa, sb, ge, be)
    return pl.pallas_call(
        _efin_body_chain,
        grid=(nb,),
        in_specs=[pl.BlockSpec(memory_space=pl.ANY)] + especs,
        out_specs=pl.BlockSpec((EBLK, D), lambda i: (i + eob, 0)),
        out_shape=jax.ShapeDtypeStruct((E, D), jnp.float32),
        input_output_aliases={0: 0},
    )(prev, e, enew, sa, sb, ge, be)


def kernel(h, e, dirn, diff, dist, edge_index, spatial_mask,
           A_w, A_b, B_w, B_b, C_w, C_b, D_w, D_b, E_w, E_b,
           g_h, b_h, g_e, b_e):
    src = edge_index[0]
    dst = edge_index[1]
    dirn_flat = jnp.concatenate(
        [dirn, jnp.zeros((N, 1), jnp.float32)], axis=1).reshape(-1)
    aux5 = (diff[:, 0], diff[:, 1], diff[:, 2], dist[:, 0],
            spatial_mask[:, 0].astype(jnp.float32))

    ah, tsrc, dh = _make_tables(h, A_w, A_b, B_w, B_b, D_w, D_b, E_w, E_b)
    g1, gd1, am1 = _gather_sc(tsrc, dh, src, dst, dirn_flat, aux5, 0, E1)
    g2, gd2, am2 = _gather_sc(tsrc, dh, src, dst, dirn_flat, aux5, E1, E2)
    nb1 = E1 // EBLK
    nb2 = E2 // EBLK
    a31 = am1.reshape(nb1, 1, EBLK)
    a32 = am2.reshape(nb2, 1, EBLK)
    enew1, hsig1, s1 = _edge_pass(e, g1, gd1, a31, C_w, C_b, 0, nb1)
    enew2, hsig2, s2 = _edge_pass(e, g2, gd2, a32, C_w, C_b, nb1, nb2)
    acc_a = _scatter_sc(hsig1, dst, 0, E1)
    acc_b = _scatter_sc(hsig2, dst, E1, E2)
    eo1 = _e_finalize(None, e, enew1, s1, s2, g_e, b_e, 0, nb1)
    eo2 = _e_finalize(eo1, e, enew2, s1, s2, g_e, b_e, nb1, nb2)
    h_out = _h_finalize(h, ah, acc_a, acc_b, g_h, b_h)
    return (h_out, eo2)
